# baseline (speedup 1.0000x reference)
import jax
import jax.numpy as jnp
from jax import lax
from jax.experimental import pallas as pl
from jax.experimental.pallas import tpu as pltpu
from jax.experimental.pallas import tpu_sc as plsc

B, N, C, H, E = 2, 2048, 1024, 16, 8
HID = 4096
HD = C // H
BN = B * N
EPS = 1e-6
NEG = -1e30

P = 2 * BN
BLK = 256
NBLK = 39
NSLOT = NBLK * BLK
CH = 512
NCH = P // CH
NW = 32
PPW = P // NW
JCH = 8
RCH = PPW // JCH


def _gelu(x):
    return 0.5 * x * (1.0 + jnp.tanh(0.7978845608028654 * (x + 0.044715 * x * x * x)))


def _ln(x):
    m = jnp.mean(x, axis=-1, keepdims=True)
    xc = x - m
    v = jnp.mean(xc * xc, axis=-1, keepdims=True)
    return xc * jax.lax.rsqrt(v + EPS)


def _mod_body(c_ref, w_ref, b_ref, o_ref):
    cc = c_ref[...]
    s = cc * jax.nn.sigmoid(cc)
    o_ref[...] = (
        jax.lax.dot_general(
            s, w_ref[...], (((1,), (1,)), ((), ())),
            preferred_element_type=jnp.float32,
        ) + b_ref[...]
    )


RB2 = 512


def _qkv_body(x_ref, mod_ref, w_ref, b_ref, o_ref):
    xn = _ln(x_ref[...])
    shift = mod_ref[0, 0, :C]
    scale = mod_ref[0, 0, C : 2 * C]
    y = (xn * (1.0 + scale) + shift).astype(jnp.bfloat16)
    o_ref[...] = (
        jax.lax.dot_general(
            y, w_ref[...], (((1,), (1,)), ((), ())),
            preferred_element_type=jnp.float32,
        ) + b_ref[...]
    ).astype(jnp.bfloat16)


BA = 1024


def _attn_body(q_ref, k_ref, v_ref, o_ref):
    ones = jnp.ones((N, HD), jnp.bfloat16)
    parts = []
    for i in range(2):
        q = q_ref[:, i * HD : (i + 1) * HD]
        k = k_ref[:, i * HD : (i + 1) * HD]
        v = v_ref[:, i * HD : (i + 1) * HD]
        s = jax.lax.dot_general(
            q, k, (((1,), (1,)), ((), ())), preferred_element_type=jnp.float32
        ) * (HD ** -0.5)
        p = jnp.exp(s - jnp.max(s, axis=1, keepdims=True)).astype(jnp.bfloat16)
        vv = jnp.concatenate([v, ones], axis=1)
        o2 = jnp.dot(p, vv, preferred_element_type=jnp.float32)
        parts.append(
            (o2[:, :HD] * (1.0 / o2[:, HD : HD + 1])).astype(jnp.bfloat16))
    o_ref[...] = jnp.concatenate(parts, axis=1)


def _proj_body(a_ref, x_ref, mod_ref, w_ref, b_ref, o_ref):
    p = (
        jax.lax.dot_general(
            a_ref[...], w_ref[...], (((1,), (1,)), ((), ())),
            preferred_element_type=jnp.float32,
        ) + b_ref[...]
    )
    g = mod_ref[0, 0, 2 * C : 3 * C]
    o_ref[...] = x_ref[...] + g * p


def _ln2_body(x2_ref, mod_ref, gw_ref, xn_ref, idx_ref, g_ref):
    xn = _ln(x2_ref[...])
    shift = mod_ref[0, 0, 3 * C : 4 * C]
    scale = mod_ref[0, 0, 4 * C : 5 * C]
    y = xn * (1.0 + scale) + shift
    xn_ref[...] = y
    yb = y.astype(jnp.bfloat16)
    logits = jax.lax.dot_general(
        yb, gw_ref[...], (((1,), (1,)), ((), ())),
        preferred_element_type=jnp.float32,
    )
    rows = logits.shape[0]
    col = jax.lax.broadcasted_iota(jnp.int32, (rows, 128), 1)
    l = jnp.where(col < E, logits, NEG)
    i1 = jnp.argmax(l, axis=1).astype(jnp.int32)
    m1 = jnp.max(l, axis=1)
    l2 = jnp.where(col == i1[:, None], NEG, l)
    i2 = jnp.argmax(l2, axis=1).astype(jnp.int32)
    m2 = jnp.max(l2, axis=1)
    e2 = jnp.exp(m2 - m1)
    g1 = (1.0 / (1.0 + e2))[:, None]
    g2 = (e2 / (1.0 + e2))[:, None]
    idx_ref[...] = jnp.where(
        col == 0, i1[:, None], jnp.where(col == 1, i2[:, None], 0)
    )
    g_ref[...] = jnp.where(col == 0, g1, jnp.where(col == 1, g2, 0.0))


def _route_body(idx_ref, dest_ref, ends_ref):
    lane = jax.lax.broadcasted_iota(jnp.int32, (BN, 128), 1)
    i1 = idx_ref[:, 0:1]
    i2 = idx_ref[:, 1:2]
    O1 = (lane == i1).astype(jnp.float32)
    O2 = (lane == i2).astype(jnp.float32)
    O = jnp.concatenate([O1, O2], axis=0)

    cnt = jnp.sum(O, axis=0, keepdims=True)
    cnt_i = cnt.astype(jnp.int32)
    padded = ((cnt_i + (BLK - 1)) // BLK) * BLK
    padded_f = padded.astype(jnp.float32)
    r128 = jax.lax.broadcasted_iota(jnp.int32, (128, 128), 0)
    c128 = jax.lax.broadcasted_iota(jnp.int32, (128, 128), 1)
    U = (r128 <= c128).astype(jnp.float32)
    ends = jnp.dot(padded_f, U, preferred_element_type=jnp.float32)
    offs = ends - padded_f
    ends_ref[...] = ends.astype(jnp.int32)

    rch = jax.lax.broadcasted_iota(jnp.int32, (CH, CH), 0)
    cch = jax.lax.broadcasted_iota(jnp.int32, (CH, CH), 1)
    L = (rch >= cch).astype(jnp.float32)

    base = offs
    for c in range(NCH):
        Oc = O[c * CH:(c + 1) * CH]
        pref = jnp.dot(L, Oc, preferred_element_type=jnp.float32)
        val = pref + base - 1.0
        destc = jnp.sum(val * Oc, axis=1, keepdims=True)
        dest_ref[c * CH:(c + 1) * CH, :] = jnp.broadcast_to(
            destc, (CH, 128)).astype(jnp.int32)
        base = base + pref[CH - 1:CH, :]


def _sc_mesh():
    return plsc.VectorSubcoreMesh(core_axis_name="c", subcore_axis_name="s")


def _sc_dispatch_body(xn_hbm, dest_hbm, xs_hbm, idx_v, rows_a, rows_b, sem_a, sem_b, sem_s):
    wid = lax.axis_index("s") * 2 + lax.axis_index("c")
    pltpu.sync_copy(dest_hbm.at[wid], idx_v)
    base = (wid % 16) * PPW
    bufs = (rows_a, rows_b)
    sems = (sem_a, sem_b)
    h = [None, None]
    h[0] = pltpu.async_copy(xn_hbm.at[pl.ds(base, RCH)], rows_a, sem_a)
    for j in range(JCH):
        cur, nxt = j % 2, (j + 1) % 2
        if j + 1 < JCH:
            h[nxt] = pltpu.async_copy(
                xn_hbm.at[pl.ds(base + (j + 1) * RCH, RCH)], bufs[nxt], sems[nxt])
        h[cur].wait()
        pltpu.async_copy(bufs[cur], xs_hbm.at[idx_v.at[j]], sem_s).wait()


def _sc_gather_body(ys_hbm, dest_hbm, yg_hbm, idx_v, rows_a, rows_b, sem_a, sem_b, sem_s):
    wid = lax.axis_index("s") * 2 + lax.axis_index("c")
    pltpu.sync_copy(dest_hbm.at[wid], idx_v)
    bufs = (rows_a, rows_b)
    sems = (sem_a, sem_b)
    h = [None, None]
    h[0] = pltpu.async_copy(ys_hbm.at[idx_v.at[0]], rows_a, sem_a)
    for j in range(JCH):
        cur, nxt = j % 2, (j + 1) % 2
        if j + 1 < JCH:
            h[nxt] = pltpu.async_copy(
                ys_hbm.at[idx_v.at[j + 1]], bufs[nxt], sems[nxt])
        h[cur].wait()
        pltpu.async_copy(
            bufs[cur], yg_hbm.at[pl.ds(wid * PPW + j * RCH, RCH)], sem_s).wait()


def _dispatch_rows(xn, dest3):
    return pl.kernel(
        _sc_dispatch_body,
        out_type=jax.ShapeDtypeStruct((NSLOT, C), jnp.float32),
        mesh=_sc_mesh(),
        scratch_types=[
            pltpu.VMEM((JCH, RCH), jnp.int32),
            pltpu.VMEM((RCH, C), jnp.float32),
            pltpu.VMEM((RCH, C), jnp.float32),
            pltpu.SemaphoreType.DMA,
            pltpu.SemaphoreType.DMA,
            pltpu.SemaphoreType.DMA,
        ],
    )(xn, dest3)


def _gather_rows(ys, dest3):
    return pl.kernel(
        _sc_gather_body,
        out_type=jax.ShapeDtypeStruct((P, C), jnp.float32),
        mesh=_sc_mesh(),
        scratch_types=[
            pltpu.VMEM((JCH, RCH), jnp.int32),
            pltpu.VMEM((RCH, C), jnp.float32),
            pltpu.VMEM((RCH, C), jnp.float32),
            pltpu.SemaphoreType.DMA,
            pltpu.SemaphoreType.DMA,
            pltpu.SemaphoreType.DMA,
        ],
    )(ys, dest3)


def _block_expert(i, ends):
    t = i * BLK
    s = jnp.int32(0)
    for e in range(E):
        s = s + (ends[e] <= t).astype(jnp.int32)
    return jnp.minimum(s, E - 1)


def _ffn1_body(ends_ref, xs_ref, w1_ref, b1_ref, h_ref):
    xb = xs_ref[...].astype(jnp.bfloat16)
    w1 = w1_ref[0].astype(jnp.bfloat16)
    h = jax.lax.dot_general(
        xb, w1, (((1,), (1,)), ((), ())),
        preferred_element_type=jnp.float32,
    ) + b1_ref[0]
    h_ref[...] = _gelu(h).astype(jnp.bfloat16)


def _ffn2_body(ends_ref, h_ref, w2_ref, b2_ref, o_ref):
    w2 = w2_ref[0].astype(jnp.bfloat16)
    o_ref[...] = jax.lax.dot_general(
        h_ref[...], w2, (((1,), (1,)), ((), ())),
        preferred_element_type=jnp.float32,
    ) + b2_ref[0]


def _combine_body(x2_ref, mod_ref, g_ref, y0_ref, y1_ref, o_ref):
    rows = x2_ref.shape[0]
    col = jax.lax.broadcasted_iota(jnp.int32, (rows, 128), 1)
    g = g_ref[...]
    g0 = jnp.sum(jnp.where(col == 0, g, 0.0), axis=1, keepdims=True)
    g1 = jnp.sum(jnp.where(col == 1, g, 0.0), axis=1, keepdims=True)
    gmlp = mod_ref[0, 0, 5 * C : 6 * C]
    o_ref[...] = x2_ref[...] + gmlp * (g0 * y0_ref[...] + g1 * y1_ref[...])


def kernel(x, c, qkv_W, qkv_b, proj_W, proj_b, gate_W, adaLN_W, adaLN_b,
           fc1_W, fc1_b, fc2_W, fc2_b):
    f32 = jnp.float32
    bf16 = jnp.bfloat16
    xf = x.reshape(BN, C)

    mod = pl.pallas_call(
        _mod_body,
        out_shape=jax.ShapeDtypeStruct((B, 6 * C), f32),
    )(c, adaLN_W, adaLN_b.reshape(1, 6 * C))
    mod3 = mod.reshape(B, 1, 6 * C)

    qkv_Wt = qkv_W.astype(bf16)
    nb2 = BN // RB2
    qkv = pl.pallas_call(
        _qkv_body,
        grid=(nb2,),
        in_specs=[
            pl.BlockSpec((RB2, C), lambda i: (i, 0)),
            pl.BlockSpec((1, 1, 6 * C), lambda i: (i * RB2 // N, 0, 0)),
            pl.BlockSpec((3 * C, C), lambda i: (0, 0)),
            pl.BlockSpec((1, 3 * C), lambda i: (0, 0)),
        ],
        out_specs=pl.BlockSpec((RB2, 3 * C), lambda i: (i, 0)),
        out_shape=jax.ShapeDtypeStruct((BN, 3 * C), bf16),
    )(xf, mod3, qkv_Wt, qkv_b.reshape(1, 3 * C))

    nba = N // BA
    attn_f = pl.pallas_call(
        _attn_body,
        grid=(B, H // 2, nba),
        in_specs=[
            pl.BlockSpec((BA, 128), lambda b, h2, r: (b * (N // BA) + r, h2)),
            pl.BlockSpec((N, 128), lambda b, h2, r: (b, 8 + h2)),
            pl.BlockSpec((N, 128), lambda b, h2, r: (b, 16 + h2)),
        ],
        out_specs=pl.BlockSpec((BA, 128), lambda b, h2, r: (b * (N // BA) + r, h2)),
        out_shape=jax.ShapeDtypeStruct((BN, C), bf16),
    )(qkv, qkv, qkv)

    proj_Wt = proj_W.astype(bf16)
    x2 = pl.pallas_call(
        _proj_body,
        grid=(nb2,),
        in_specs=[
            pl.BlockSpec((RB2, C), lambda i: (i, 0)),
            pl.BlockSpec((RB2, C), lambda i: (i, 0)),
            pl.BlockSpec((1, 1, 6 * C), lambda i: (i * RB2 // N, 0, 0)),
            pl.BlockSpec((C, C), lambda i: (0, 0)),
            pl.BlockSpec((1, C), lambda i: (0, 0)),
        ],
        out_specs=pl.BlockSpec((RB2, C), lambda i: (i, 0)),
        out_shape=jax.ShapeDtypeStruct((BN, C), f32),
    )(attn_f, xf, mod3, proj_Wt, proj_b.reshape(1, C))

    gate_Wp = jnp.zeros((128, C), f32).at[:E].set(gate_W).astype(bf16)
    xn, idx, gates = pl.pallas_call(
        _ln2_body,
        grid=(nb2,),
        in_specs=[
            pl.BlockSpec((RB2, C), lambda i: (i, 0)),
            pl.BlockSpec((1, 1, 6 * C), lambda i: (i * RB2 // N, 0, 0)),
            pl.BlockSpec((128, C), lambda i: (0, 0)),
        ],
        out_specs=[
            pl.BlockSpec((RB2, C), lambda i: (i, 0)),
            pl.BlockSpec((RB2, 128), lambda i: (i, 0)),
            pl.BlockSpec((RB2, 128), lambda i: (i, 0)),
        ],
        out_shape=[
            jax.ShapeDtypeStruct((BN, C), f32),
            jax.ShapeDtypeStruct((BN, 128), jnp.int32),
            jax.ShapeDtypeStruct((BN, 128), f32),
        ],
    )(x2, mod3, gate_Wp)

    dest_b, ends_b = pl.pallas_call(
        _route_body,
        out_shape=[
            jax.ShapeDtypeStruct((P, 128), jnp.int32),
            jax.ShapeDtypeStruct((1, 128), jnp.int32),
        ],
    )(idx)
    dest3 = dest_b[:, 0].reshape(NW, JCH, RCH)
    ends = ends_b[0, :E]

    xs = _dispatch_rows(xn, dest3)

    gs1 = pltpu.PrefetchScalarGridSpec(
        num_scalar_prefetch=1,
        grid=(2, NBLK),
        in_specs=[
            pl.BlockSpec((BLK, C), lambda j, i, ends: (i, 0)),
            pl.BlockSpec((1, HID // 2, C),
                         lambda j, i, ends: (_block_expert(i, ends), j, 0)),
            pl.BlockSpec((1, 1, HID // 2),
                         lambda j, i, ends: (_block_expert(i, ends), 0, j)),
        ],
        out_specs=pl.BlockSpec((BLK, HID // 2), lambda j, i, ends: (i, j)),
    )
    hmid = pl.pallas_call(
        _ffn1_body,
        grid_spec=gs1,
        out_shape=jax.ShapeDtypeStruct((NSLOT, HID), bf16),
    )(ends, xs, fc1_W, fc1_b.reshape(E, 1, HID))

    gs2 = pltpu.PrefetchScalarGridSpec(
        num_scalar_prefetch=1,
        grid=(2, NBLK),
        in_specs=[
            pl.BlockSpec((BLK, HID), lambda j, i, ends: (i, 0)),
            pl.BlockSpec((1, C // 2, HID),
                         lambda j, i, ends: (_block_expert(i, ends), j, 0)),
            pl.BlockSpec((1, 1, C // 2),
                         lambda j, i, ends: (_block_expert(i, ends), 0, j)),
        ],
        out_specs=pl.BlockSpec((BLK, C // 2), lambda j, i, ends: (i, j)),
    )
    ys = pl.pallas_call(
        _ffn2_body,
        grid_spec=gs2,
        out_shape=jax.ShapeDtypeStruct((NSLOT, C), f32),
    )(ends, hmid, fc2_W, fc2_b.reshape(E, 1, C))

    yg = _gather_rows(ys, dest3)

    out = pl.pallas_call(
        _combine_body,
        grid=(nb2,),
        in_specs=[
            pl.BlockSpec((RB2, C), lambda i: (i, 0)),
            pl.BlockSpec((1, 1, 6 * C), lambda i: (i * RB2 // N, 0, 0)),
            pl.BlockSpec((RB2, 128), lambda i: (i, 0)),
            pl.BlockSpec((RB2, C), lambda i: (i, 0)),
            pl.BlockSpec((RB2, C), lambda i: (i + BN // RB2, 0)),
        ],
        out_specs=pl.BlockSpec((RB2, C), lambda i: (i, 0)),
        out_shape=jax.ShapeDtypeStruct((BN, C), f32),
    )(x2, mod3, gates, yg, yg)

    return out.reshape(B, N, C)


# speedup vs baseline: 1.0609x; 1.0609x over previous
import jax
import jax.numpy as jnp
from jax import lax
from jax.experimental import pallas as pl
from jax.experimental.pallas import tpu as pltpu
from jax.experimental.pallas import tpu_sc as plsc

B, N, C, H, E = 2, 2048, 1024, 16, 8
HID = 4096
HD = C // H
BN = B * N
EPS = 1e-6
NEG = -1e30

P = 2 * BN
BLK = 256
NBLK = 39
NSLOT = NBLK * BLK
CH = 512
NCH = P // CH
NW = 32
PPW = P // NW
JCH = 8
RCH = PPW // JCH


def _gelu(x):
    return 0.5 * x * (1.0 + jnp.tanh(0.7978845608028654 * (x + 0.044715 * x * x * x)))


def _ln(x):
    m = jnp.mean(x, axis=-1, keepdims=True)
    xc = x - m
    v = jnp.mean(xc * xc, axis=-1, keepdims=True)
    return xc * jax.lax.rsqrt(v + EPS)


def _mod_body(c_ref, w_ref, b_ref, o_ref):
    cc = c_ref[...]
    s = cc * jax.nn.sigmoid(cc)
    o_ref[...] = (
        jax.lax.dot_general(
            s, w_ref[...], (((1,), (1,)), ((), ())),
            preferred_element_type=jnp.float32,
        ) + b_ref[...]
    )


RB2 = 512


def _qkv_body(x_ref, mod_ref, w_ref, b_ref, o_ref):
    xn = _ln(x_ref[...])
    shift = mod_ref[0, 0, :C]
    scale = mod_ref[0, 0, C : 2 * C]
    y = (xn * (1.0 + scale) + shift).astype(jnp.bfloat16)
    o_ref[...] = (
        jax.lax.dot_general(
            y, w_ref[...], (((1,), (1,)), ((), ())),
            preferred_element_type=jnp.float32,
        ) + b_ref[...]
    ).astype(jnp.bfloat16)


BA = 1024


def _attn_body(q_ref, k_ref, v_ref, o_ref):
    ones = jnp.ones((N, HD), jnp.bfloat16)
    parts = []
    for i in range(2):
        q = q_ref[:, i * HD : (i + 1) * HD]
        k = k_ref[:, i * HD : (i + 1) * HD]
        v = v_ref[:, i * HD : (i + 1) * HD]
        s = jax.lax.dot_general(
            q, k, (((1,), (1,)), ((), ())), preferred_element_type=jnp.float32
        ) * (HD ** -0.5)
        p = jnp.exp(s - jnp.max(s, axis=1, keepdims=True)).astype(jnp.bfloat16)
        vv = jnp.concatenate([v, ones], axis=1)
        o2 = jnp.dot(p, vv, preferred_element_type=jnp.float32)
        parts.append(
            (o2[:, :HD] * (1.0 / o2[:, HD : HD + 1])).astype(jnp.bfloat16))
    o_ref[...] = jnp.concatenate(parts, axis=1)


def _proj_body(a_ref, x_ref, mod_ref, w_ref, b_ref, o_ref):
    p = (
        jax.lax.dot_general(
            a_ref[...], w_ref[...], (((1,), (1,)), ((), ())),
            preferred_element_type=jnp.float32,
        ) + b_ref[...]
    )
    g = mod_ref[0, 0, 2 * C : 3 * C]
    o_ref[...] = x_ref[...] + g * p


def _ln2_body(x2_ref, mod_ref, gw_ref, xn_ref, idx_ref, g_ref):
    xn = _ln(x2_ref[...])
    shift = mod_ref[0, 0, 3 * C : 4 * C]
    scale = mod_ref[0, 0, 4 * C : 5 * C]
    y = xn * (1.0 + scale) + shift
    xn_ref[...] = y
    yb = y.astype(jnp.bfloat16)
    logits = jax.lax.dot_general(
        yb, gw_ref[...], (((1,), (1,)), ((), ())),
        preferred_element_type=jnp.float32,
    )
    rows = logits.shape[0]
    col = jax.lax.broadcasted_iota(jnp.int32, (rows, 128), 1)
    l = jnp.where(col < E, logits, NEG)
    i1 = jnp.argmax(l, axis=1).astype(jnp.int32)
    m1 = jnp.max(l, axis=1)
    l2 = jnp.where(col == i1[:, None], NEG, l)
    i2 = jnp.argmax(l2, axis=1).astype(jnp.int32)
    m2 = jnp.max(l2, axis=1)
    e2 = jnp.exp(m2 - m1)
    g1 = (1.0 / (1.0 + e2))[:, None]
    g2 = (e2 / (1.0 + e2))[:, None]
    idx_ref[...] = jnp.where(
        col == 0, i1[:, None], jnp.where(col == 1, i2[:, None], 0)
    )
    g_ref[...] = jnp.where(col == 0, g1, jnp.where(col == 1, g2, 0.0))


def _route_body(idx_ref, dest_ref, ends_ref):
    lane = jax.lax.broadcasted_iota(jnp.int32, (BN, 128), 1)
    i1 = idx_ref[:, 0:1]
    i2 = idx_ref[:, 1:2]
    O1 = (lane == i1).astype(jnp.float32)
    O2 = (lane == i2).astype(jnp.float32)
    O = jnp.concatenate([O1, O2], axis=0)

    cnt = jnp.sum(O, axis=0, keepdims=True)
    cnt_i = cnt.astype(jnp.int32)
    padded = ((cnt_i + (BLK - 1)) // BLK) * BLK
    padded_f = padded.astype(jnp.float32)
    r128 = jax.lax.broadcasted_iota(jnp.int32, (128, 128), 0)
    c128 = jax.lax.broadcasted_iota(jnp.int32, (128, 128), 1)
    U = (r128 <= c128).astype(jnp.float32)
    ends = jnp.dot(padded_f, U, preferred_element_type=jnp.float32)
    offs = ends - padded_f
    ends_ref[...] = ends.astype(jnp.int32)

    rch = jax.lax.broadcasted_iota(jnp.int32, (CH, CH), 0)
    cch = jax.lax.broadcasted_iota(jnp.int32, (CH, CH), 1)
    L = (rch >= cch).astype(jnp.float32)

    base = offs
    for c in range(NCH):
        Oc = O[c * CH:(c + 1) * CH]
        pref = jnp.dot(L, Oc, preferred_element_type=jnp.float32)
        val = pref + base - 1.0
        destc = jnp.sum(val * Oc, axis=1, keepdims=True)
        dest_ref[c * CH:(c + 1) * CH, :] = jnp.broadcast_to(
            destc, (CH, 128)).astype(jnp.int32)
        base = base + pref[CH - 1:CH, :]


def _sc_mesh():
    return plsc.VectorSubcoreMesh(core_axis_name="c", subcore_axis_name="s")


def _sc_dispatch_body(xn_hbm, dest_hbm, xs_hbm, idx_v, rows_a, rows_b, sem_a, sem_b, sem_s):
    wid = lax.axis_index("s") * 2 + lax.axis_index("c")
    pltpu.sync_copy(dest_hbm.at[wid], idx_v)
    base = (wid % 16) * PPW
    bufs = (rows_a, rows_b)
    sems = (sem_a, sem_b)
    h = [None, None]
    h[0] = pltpu.async_copy(xn_hbm.at[pl.ds(base, RCH)], rows_a, sem_a)
    for j in range(JCH):
        cur, nxt = j % 2, (j + 1) % 2
        if j + 1 < JCH:
            h[nxt] = pltpu.async_copy(
                xn_hbm.at[pl.ds(base + (j + 1) * RCH, RCH)], bufs[nxt], sems[nxt])
        h[cur].wait()
        pltpu.async_copy(bufs[cur], xs_hbm.at[idx_v.at[j]], sem_s).wait()


def _sc_gather_body(ys_hbm, dest_hbm, yg_hbm, idx_v, rows_a, rows_b, sem_a, sem_b, sem_s):
    wid = lax.axis_index("s") * 2 + lax.axis_index("c")
    pltpu.sync_copy(dest_hbm.at[wid], idx_v)
    bufs = (rows_a, rows_b)
    sems = (sem_a, sem_b)
    h = [None, None]
    h[0] = pltpu.async_copy(ys_hbm.at[idx_v.at[0]], rows_a, sem_a)
    for j in range(JCH):
        cur, nxt = j % 2, (j + 1) % 2
        if j + 1 < JCH:
            h[nxt] = pltpu.async_copy(
                ys_hbm.at[idx_v.at[j + 1]], bufs[nxt], sems[nxt])
        h[cur].wait()
        pltpu.async_copy(
            bufs[cur], yg_hbm.at[pl.ds(wid * PPW + j * RCH, RCH)], sem_s).wait()


def _dispatch_rows(xn, dest3):
    return pl.kernel(
        _sc_dispatch_body,
        out_type=jax.ShapeDtypeStruct((NSLOT, C), jnp.float32),
        mesh=_sc_mesh(),
        scratch_types=[
            pltpu.VMEM((JCH, RCH), jnp.int32),
            pltpu.VMEM((RCH, C), jnp.float32),
            pltpu.VMEM((RCH, C), jnp.float32),
            pltpu.SemaphoreType.DMA,
            pltpu.SemaphoreType.DMA,
            pltpu.SemaphoreType.DMA,
        ],
    )(xn, dest3)


def _gather_rows(ys, dest3):
    return pl.kernel(
        _sc_gather_body,
        out_type=jax.ShapeDtypeStruct((P, C), jnp.float32),
        mesh=_sc_mesh(),
        scratch_types=[
            pltpu.VMEM((JCH, RCH), jnp.int32),
            pltpu.VMEM((RCH, C), jnp.float32),
            pltpu.VMEM((RCH, C), jnp.float32),
            pltpu.SemaphoreType.DMA,
            pltpu.SemaphoreType.DMA,
            pltpu.SemaphoreType.DMA,
        ],
    )(ys, dest3)


def _block_expert(i, ends):
    t = i * BLK
    s = jnp.int32(0)
    for e in range(E):
        s = s + (ends[e] <= t).astype(jnp.int32)
    return jnp.minimum(s, E - 1)


def _ffn1_body(ends_ref, xs_ref, w1_ref, b1_ref, h_ref):
    xb = xs_ref[...].astype(jnp.bfloat16)
    w1 = w1_ref[0].astype(jnp.bfloat16)
    h = jax.lax.dot_general(
        xb, w1, (((1,), (1,)), ((), ())),
        preferred_element_type=jnp.float32,
    ) + b1_ref[0]
    h_ref[...] = _gelu(h).astype(jnp.bfloat16)


def _ffn2_body(ends_ref, h_ref, w2_ref, b2_ref, o_ref):
    w2 = w2_ref[0].astype(jnp.bfloat16)
    o_ref[...] = jax.lax.dot_general(
        h_ref[...], w2, (((1,), (1,)), ((), ())),
        preferred_element_type=jnp.float32,
    ) + b2_ref[0]


def _combine_body(x2_ref, mod_ref, g_ref, y0_ref, y1_ref, o_ref):
    rows = x2_ref.shape[0]
    col = jax.lax.broadcasted_iota(jnp.int32, (rows, 128), 1)
    g = g_ref[...]
    g0 = jnp.sum(jnp.where(col == 0, g, 0.0), axis=1, keepdims=True)
    g1 = jnp.sum(jnp.where(col == 1, g, 0.0), axis=1, keepdims=True)
    gmlp = mod_ref[0, 0, 5 * C : 6 * C]
    o_ref[...] = x2_ref[...] + gmlp * (g0 * y0_ref[...] + g1 * y1_ref[...])


def kernel(x, c, qkv_W, qkv_b, proj_W, proj_b, gate_W, adaLN_W, adaLN_b,
           fc1_W, fc1_b, fc2_W, fc2_b):
    f32 = jnp.float32
    bf16 = jnp.bfloat16
    xf = x.reshape(BN, C)

    mod = pl.pallas_call(
        _mod_body,
        out_shape=jax.ShapeDtypeStruct((B, 6 * C), f32),
    )(c, adaLN_W, adaLN_b.reshape(1, 6 * C))
    mod3 = mod.reshape(B, 1, 6 * C)

    qkv_Wt = qkv_W.astype(bf16)
    nb2 = BN // RB2
    qkv = pl.pallas_call(
        _qkv_body,
        grid=(nb2,),
        in_specs=[
            pl.BlockSpec((RB2, C), lambda i: (i, 0)),
            pl.BlockSpec((1, 1, 6 * C), lambda i: (i * RB2 // N, 0, 0)),
            pl.BlockSpec((3 * C, C), lambda i: (0, 0)),
            pl.BlockSpec((1, 3 * C), lambda i: (0, 0)),
        ],
        out_specs=pl.BlockSpec((RB2, 3 * C), lambda i: (i, 0)),
        out_shape=jax.ShapeDtypeStruct((BN, 3 * C), bf16),
    )(xf, mod3, qkv_Wt, qkv_b.reshape(1, 3 * C))

    nba = N // BA
    attn_f = pl.pallas_call(
        _attn_body,
        grid=(B, H // 2, nba),
        in_specs=[
            pl.BlockSpec((BA, 128), lambda b, h2, r: (b * (N // BA) + r, h2)),
            pl.BlockSpec((N, 128), lambda b, h2, r: (b, 8 + h2)),
            pl.BlockSpec((N, 128), lambda b, h2, r: (b, 16 + h2)),
        ],
        out_specs=pl.BlockSpec((BA, 128), lambda b, h2, r: (b * (N // BA) + r, h2)),
        out_shape=jax.ShapeDtypeStruct((BN, C), bf16),
    )(qkv, qkv, qkv)

    proj_Wt = proj_W.astype(bf16)
    x2 = pl.pallas_call(
        _proj_body,
        grid=(nb2,),
        in_specs=[
            pl.BlockSpec((RB2, C), lambda i: (i, 0)),
            pl.BlockSpec((RB2, C), lambda i: (i, 0)),
            pl.BlockSpec((1, 1, 6 * C), lambda i: (i * RB2 // N, 0, 0)),
            pl.BlockSpec((C, C), lambda i: (0, 0)),
            pl.BlockSpec((1, C), lambda i: (0, 0)),
        ],
        out_specs=pl.BlockSpec((RB2, C), lambda i: (i, 0)),
        out_shape=jax.ShapeDtypeStruct((BN, C), f32),
    )(attn_f, xf, mod3, proj_Wt, proj_b.reshape(1, C))

    gate_Wp = jnp.zeros((128, C), f32).at[:E].set(gate_W).astype(bf16)
    xn, idx, gates = pl.pallas_call(
        _ln2_body,
        grid=(nb2,),
        in_specs=[
            pl.BlockSpec((RB2, C), lambda i: (i, 0)),
            pl.BlockSpec((1, 1, 6 * C), lambda i: (i * RB2 // N, 0, 0)),
            pl.BlockSpec((128, C), lambda i: (0, 0)),
        ],
        out_specs=[
            pl.BlockSpec((RB2, C), lambda i: (i, 0)),
            pl.BlockSpec((RB2, 128), lambda i: (i, 0)),
            pl.BlockSpec((RB2, 128), lambda i: (i, 0)),
        ],
        out_shape=[
            jax.ShapeDtypeStruct((BN, C), f32),
            jax.ShapeDtypeStruct((BN, 128), jnp.int32),
            jax.ShapeDtypeStruct((BN, 128), f32),
        ],
    )(x2, mod3, gate_Wp)

    dest_b, ends_b = pl.pallas_call(
        _route_body,
        out_shape=[
            jax.ShapeDtypeStruct((P, 128), jnp.int32),
            jax.ShapeDtypeStruct((1, 128), jnp.int32),
        ],
    )(idx)
    dest3 = dest_b[:, 0].reshape(NW, JCH, RCH)
    ends = ends_b[0, :E]

    xs = _dispatch_rows(xn, dest3)

    gs1 = pltpu.PrefetchScalarGridSpec(
        num_scalar_prefetch=1,
        grid=(NBLK,),
        in_specs=[
            pl.BlockSpec((BLK, C), lambda i, ends: (i, 0)),
            pl.BlockSpec((1, HID, C),
                         lambda i, ends: (_block_expert(i, ends), 0, 0)),
            pl.BlockSpec((1, 1, HID),
                         lambda i, ends: (_block_expert(i, ends), 0, 0)),
        ],
        out_specs=pl.BlockSpec((BLK, HID), lambda i, ends: (i, 0)),
    )
    hmid = pl.pallas_call(
        _ffn1_body,
        grid_spec=gs1,
        out_shape=jax.ShapeDtypeStruct((NSLOT, HID), bf16),
    )(ends, xs, fc1_W, fc1_b.reshape(E, 1, HID))

    gs2 = pltpu.PrefetchScalarGridSpec(
        num_scalar_prefetch=1,
        grid=(NBLK,),
        in_specs=[
            pl.BlockSpec((BLK, HID), lambda i, ends: (i, 0)),
            pl.BlockSpec((1, C, HID),
                         lambda i, ends: (_block_expert(i, ends), 0, 0)),
            pl.BlockSpec((1, 1, C),
                         lambda i, ends: (_block_expert(i, ends), 0, 0)),
        ],
        out_specs=pl.BlockSpec((BLK, C), lambda i, ends: (i, 0)),
    )
    ys = pl.pallas_call(
        _ffn2_body,
        grid_spec=gs2,
        out_shape=jax.ShapeDtypeStruct((NSLOT, C), f32),
    )(ends, hmid, fc2_W, fc2_b.reshape(E, 1, C))

    yg = _gather_rows(ys, dest3)

    out = pl.pallas_call(
        _combine_body,
        grid=(nb2,),
        in_specs=[
            pl.BlockSpec((RB2, C), lambda i: (i, 0)),
            pl.BlockSpec((1, 1, 6 * C), lambda i: (i * RB2 // N, 0, 0)),
            pl.BlockSpec((RB2, 128), lambda i: (i, 0)),
            pl.BlockSpec((RB2, C), lambda i: (i, 0)),
            pl.BlockSpec((RB2, C), lambda i: (i + BN // RB2, 0)),
        ],
        out_specs=pl.BlockSpec((RB2, C), lambda i: (i, 0)),
        out_shape=jax.ShapeDtypeStruct((BN, C), f32),
    )(x2, mod3, gates, yg, yg)

    return out.reshape(B, N, C)
